# baseline (speedup 1.0000x reference)
import functools

import jax
import jax.numpy as jnp
from jax import lax
from jax.experimental import pallas as pl
from jax.experimental.pallas import tpu as pltpu
from jax.experimental.pallas import tpu_sc as plsc

_D_MODEL = 2048
_L = 16


def _sc_gather(table, idx, n_tokens):
    info = plsc.get_sparse_core_info()
    nc, ns = info.num_cores, info.num_subcores
    nw = nc * ns
    per_w = n_tokens // nw
    ngroups = per_w // _L

    mesh = plsc.VectorSubcoreMesh(core_axis_name="c", subcore_axis_name="s")

    @functools.partial(
        pl.kernel,
        out_type=jax.ShapeDtypeStruct((n_tokens, _D_MODEL), jnp.float32),
        mesh=mesh,
        scratch_types=[
            pltpu.VMEM((per_w,), jnp.int32),
            pltpu.SemaphoreType.DMA,
        ],
    )
    def body(table_hbm, idx_hbm, out_hbm, idx_v, sem):
        wid = lax.axis_index("s") * nc + lax.axis_index("c")
        base = wid * per_w
        pltpu.sync_copy(idx_hbm.at[pl.ds(base, per_w)], idx_v)

        def group(g, carry):
            vec = idx_v[pl.ds(g * _L, _L)]
            for j in range(_L):
                r = vec[j]
                pltpu.async_copy(
                    table_hbm.at[pl.ds(r, 1)],
                    out_hbm.at[pl.ds(base + g * _L + j, 1)],
                    sem,
                )
            return carry

        lax.fori_loop(0, ngroups, group, 0)

        def drain(g, carry):
            pltpu.make_async_copy(
                table_hbm.at[pl.ds(0, 1)], out_hbm.at[pl.ds(base, 1)], sem
            ).wait()
            return carry

        lax.fori_loop(0, per_w, drain, 0)

    return body(table, idx)


def kernel(input_ids, embed_table):
    b, s = input_ids.shape
    idx = input_ids.reshape(-1).astype(jnp.int32)
    flat = _sc_gather(embed_table, idx, b * s)
    return flat.reshape(b, s, _D_MODEL)


# speedup vs baseline: 39.6696x; 39.6696x over previous
import functools

import jax
import jax.numpy as jnp
from jax import lax
from jax.experimental import pallas as pl
from jax.experimental.pallas import tpu as pltpu
from jax.experimental.pallas import tpu_sc as plsc

_D_MODEL = 2048
_CHUNK = 8
_NBUF = 6
_LOOKAHEAD = 4


def _sc_gather(table, idx, n_tokens):
    info = plsc.get_sparse_core_info()
    nc, ns = info.num_cores, info.num_subcores
    nw = nc * ns
    per_w = n_tokens // nw
    nchunks = per_w // _CHUNK

    mesh = plsc.VectorSubcoreMesh(core_axis_name="c", subcore_axis_name="s")

    @functools.partial(
        pl.kernel,
        out_type=jax.ShapeDtypeStruct((n_tokens, _D_MODEL), jnp.float32),
        mesh=mesh,
        scratch_types=[
            pltpu.VMEM((per_w,), jnp.int32),
        ]
        + [pltpu.VMEM((_CHUNK, _D_MODEL), jnp.float32)] * _NBUF
        + [pltpu.SemaphoreType.DMA] * (2 * _NBUF),
    )
    def body(table_hbm, idx_hbm, out_hbm, idx_v, *bufs_sems):
        rows = bufs_sems[:_NBUF]
        gsem = bufs_sems[_NBUF : 2 * _NBUF]
        osem = bufs_sems[2 * _NBUF :]

        wid = lax.axis_index("s") * nc + lax.axis_index("c")
        base = wid * per_w
        pltpu.sync_copy(idx_hbm.at[pl.ds(base, per_w)], idx_v)

        def gather_start(chunk, b):
            pltpu.async_copy(
                table_hbm.at[idx_v.at[pl.ds(chunk * _CHUNK, _CHUNK)]],
                rows[b],
                gsem[b],
            )

        def gather_wait(b):
            pltpu.make_async_copy(
                table_hbm.at[idx_v.at[pl.ds(0, _CHUNK)]], rows[b], gsem[b]
            ).wait()

        def out_start(chunk, b):
            pltpu.async_copy(
                rows[b], out_hbm.at[pl.ds(base + chunk * _CHUNK, _CHUNK)], osem[b]
            )

        def out_wait(b):
            pltpu.make_async_copy(
                rows[b], out_hbm.at[pl.ds(base, _CHUNK)], osem[b]
            ).wait()

        for k in range(_LOOKAHEAD):
            gather_start(k, k % _NBUF)

        def outer(t, carry):
            for j in range(_NBUF):
                i = _NBUF * t + j
                gather_wait(j)
                bg = (j + _LOOKAHEAD) % _NBUF

                @pl.when(i + _LOOKAHEAD < nchunks)
                def _():
                    @pl.when(i + _LOOKAHEAD >= _NBUF)
                    def _():
                        out_wait(bg)

                    gather_start(i + _LOOKAHEAD, bg)

                out_start(i, j)
            return carry

        lax.fori_loop(0, nchunks // _NBUF, outer, 0)
        for i in range(_NBUF * (nchunks // _NBUF), nchunks):
            gather_wait(i % _NBUF)
            out_start(i, i % _NBUF)
        for i in range(max(nchunks - _NBUF, 0), nchunks):
            out_wait(i % _NBUF)

    return body(table, idx)


def kernel(input_ids, embed_table):
    b, s = input_ids.shape
    idx = input_ids.reshape(-1).astype(jnp.int32)
    flat = _sc_gather(embed_table, idx, b * s)
    return flat.reshape(b, s, _D_MODEL)


# speedup vs baseline: 39.7852x; 1.0029x over previous
import functools

import jax
import jax.numpy as jnp
from jax import lax
from jax.experimental import pallas as pl
from jax.experimental.pallas import tpu as pltpu
from jax.experimental.pallas import tpu_sc as plsc

_D_MODEL = 2048
_CHUNK = 8
_NBUF = 4


def _sc_gather(table, idx, n_tokens):
    info = plsc.get_sparse_core_info()
    nc, ns = info.num_cores, info.num_subcores
    nw = nc * ns
    per_w = n_tokens // nw
    nchunks = per_w // _CHUNK

    mesh = plsc.VectorSubcoreMesh(core_axis_name="c", subcore_axis_name="s")

    @functools.partial(
        pl.kernel,
        out_type=jax.ShapeDtypeStruct((n_tokens, _D_MODEL), jnp.float32),
        mesh=mesh,
        scratch_types=[
            pltpu.VMEM((per_w,), jnp.int32),
        ]
        + [pltpu.VMEM((_CHUNK, _D_MODEL), jnp.float32)] * _NBUF
        + [pltpu.SemaphoreType.DMA] * (2 * _NBUF),
    )
    def body(table_hbm, idx_hbm, out_hbm, idx_v, *bufs_sems):
        rows = bufs_sems[:_NBUF]
        gsem = bufs_sems[_NBUF : 2 * _NBUF]
        osem = bufs_sems[2 * _NBUF :]

        wid = lax.axis_index("s") * nc + lax.axis_index("c")
        base = wid * per_w
        pltpu.sync_copy(idx_hbm.at[pl.ds(base, per_w)], idx_v)

        def gather_start(chunk, b):
            pltpu.async_copy(
                table_hbm.at[idx_v.at[pl.ds(chunk * _CHUNK, _CHUNK)]],
                rows[b],
                gsem[b],
            )

        def gather_wait(b):
            pltpu.make_async_copy(
                table_hbm.at[idx_v.at[pl.ds(0, _CHUNK)]], rows[b], gsem[b]
            ).wait()

        def out_start(chunk, b):
            pltpu.async_copy(
                rows[b], out_hbm.at[pl.ds(base + chunk * _CHUNK, _CHUNK)], osem[b]
            )

        def out_wait(b):
            pltpu.make_async_copy(
                rows[b], out_hbm.at[pl.ds(base, _CHUNK)], osem[b]
            ).wait()

        gather_start(0, 0)
        gather_start(1, 1)

        def outer(t, carry):
            for j in range(_NBUF):
                i = _NBUF * t + j
                gather_wait(j)
                bg = (j + 2) % _NBUF

                @pl.when(i + 2 < nchunks)
                def _():
                    @pl.when(i >= 2)
                    def _():
                        out_wait(bg)

                    gather_start(i + 2, bg)

                out_start(i, j)
            return carry

        lax.fori_loop(0, nchunks // _NBUF, outer, 0)
        for b in range(_NBUF):
            out_wait(b)

    return body(table, idx)


def kernel(input_ids, embed_table):
    b, s = input_ids.shape
    idx = input_ids.reshape(-1).astype(jnp.int32)
    flat = _sc_gather(embed_table, idx, b * s)
    return flat.reshape(b, s, _D_MODEL)
